# baseline (speedup 1.0000x reference)
import jax
import jax.numpy as jnp
from jax import lax
from jax.experimental import pallas as pl
from jax.experimental.pallas import tpu as pltpu
from jax.experimental.pallas import tpu_sc as plsc

_B, _L, _D = 16384, 200, 32
_NC, _NS = 2, 16
_NW = _NC * _NS
_LT = _L // 8
_BT = _B // 128
_NU = _LT * _BT // _NW
_NBUF = 2
_NGROUPS = _NU // _NBUF


def _emb_body(x_hbm, table_hbm, o_hbm,
              idx0, idx1, rows0, rows1, tout, gs0, gs1, ws):
    idx = (idx0, idx1)
    rows = (rows0, rows1)
    gsem = (gs0, gs1)
    wid = lax.axis_index("s") * _NC + lax.axis_index("c")
    u0 = wid * _NU
    cbase = [lax.iota(jnp.int32, 16) + 16 * j for j in range(8)]

    def unit_lt_bt(u):
        uid = u0 + u
        return uid // _BT, uid % _BT

    def fire(b, u):
        lt, bt = unit_lt_bt(u)
        pltpu.sync_copy(x_hbm.at[lt, bt], idx[b])
        for s in range(8):
            pltpu.async_copy(table_hbm.at[idx[b].at[s]], rows[b].at[s],
                             gsem[b])

    def wait_gathers(b):
        for s in range(8):
            pltpu.make_async_copy(table_hbm.at[idx[b].at[s]], rows[b].at[s],
                                  gsem[b]).wait()

    def write_descs(u):
        lt, bt = unit_lt_bt(u)
        descs = []
        for s in range(8):
            for dt in range(4):
                descs.append(pltpu.make_async_copy(
                    tout.at[s, pl.ds(8 * dt, 8)],
                    o_hbm.at[8 * lt + s, dt, bt], ws))
        return descs

    def transpose(b):
        for s in range(8):
            r_s = rows[b].at[s]

            def tbody(k, carry, s=s, r_s=r_s):
                for dd in range(4):
                    d = k * 4 + dd
                    dvec = jnp.zeros((16,), jnp.int32) + d
                    for j in range(8):
                        v = plsc.load_gather(r_s, [cbase[j], dvec])
                        tout[s, d, pl.ds(16 * j, 16)] = v
                return carry

            lax.fori_loop(0, 8, tbody, 0)

    for b in range(_NBUF):
        fire(b, b)

    def body(i, carry):
        for b in range(_NBUF):
            u = i * _NBUF + b
            wait_gathers(b)

            @pl.when(u > 0)
            def _():
                for dsc in write_descs(u - 1):
                    dsc.wait()

            transpose(b)
            for dsc in write_descs(u):
                dsc.start()

            @pl.when(i < _NGROUPS - 1)
            def _():
                fire(b, u + _NBUF)

        return carry

    lax.fori_loop(0, _NGROUPS, body, 0)
    for dsc in write_descs(_NU - 1):
        dsc.wait()


@jax.jit
def _embed(x, table):
    mesh = plsc.VectorSubcoreMesh(core_axis_name="c", subcore_axis_name="s")
    f = pl.kernel(
        _emb_body,
        out_type=jax.ShapeDtypeStruct((_L, 4, 128, 8, 128), jnp.float32),
        mesh=mesh,
        scratch_types=[
            pltpu.VMEM((8, 128), jnp.int32),
            pltpu.VMEM((8, 128), jnp.int32),
            pltpu.VMEM((8, 128, _D), jnp.float32),
            pltpu.VMEM((8, 128, _D), jnp.float32),
            pltpu.VMEM((8, _D, 128), jnp.float32),
            pltpu.SemaphoreType.DMA,
            pltpu.SemaphoreType.DMA,
            pltpu.SemaphoreType.DMA,
        ],
        compiler_params=pltpu.CompilerParams(use_tc_tiling_on_sc=False,
                                             needs_layout_passes=False),
    )
    return f(x, table)


def kernel(ys, table):
    x = jnp.transpose(
        jnp.reshape(jnp.transpose(ys.astype(jnp.int32), (1, 0)),
                    (_LT, 8, _BT, 128)),
        (0, 2, 1, 3))
    o = _embed(x, table)
    return jnp.reshape(jnp.transpose(o, (2, 4, 0, 1, 3)), (_B, _L, _D))


# speedup vs baseline: 1.2076x; 1.2076x over previous
import jax
import jax.numpy as jnp
from jax import lax
from jax.experimental import pallas as pl
from jax.experimental.pallas import tpu as pltpu
from jax.experimental.pallas import tpu_sc as plsc

_B, _L, _D = 16384, 200, 32
_NC, _NS = 2, 16
_NW = _NC * _NS
_LT = _L // 8
_BT = _B // 128
_NU = _LT * _BT // _NW
_NBUF = 2
_NGROUPS = _NU // _NBUF


def _emb_body(x_hbm, table_hbm, o_hbm,
              idx0, idx1, rows0, rows1, tout, gs0, gs1, ws):
    idx = (idx0, idx1)
    rows = (rows0, rows1)
    gsem = (gs0, gs1)
    wid = lax.axis_index("s") * _NC + lax.axis_index("c")
    u0 = wid * _NU
    lane = lax.iota(jnp.int32, 16)
    dbase0 = lane * 128
    dbase1 = dbase0 + 16 * 128

    def unit_lt_bt(u):
        uid = u0 + u
        return uid // _BT, uid % _BT

    def fire(b, u):
        lt, bt = unit_lt_bt(u)
        pltpu.sync_copy(x_hbm.at[lt, bt], idx[b])
        for s in range(8):
            pltpu.async_copy(table_hbm.at[idx[b].at[s]], rows[b].at[s],
                             gsem[b])

    def wait_gathers(b):
        for s in range(8):
            pltpu.make_async_copy(table_hbm.at[idx[b].at[s]], rows[b].at[s],
                                  gsem[b]).wait()

    def write_descs(u):
        lt, bt = unit_lt_bt(u)
        descs = []
        for s in range(8):
            for dt in range(4):
                descs.append(pltpu.make_async_copy(
                    tout.at[s, pl.ds(1024 * dt, 1024)],
                    o_hbm.at[8 * lt + s, dt, bt], ws))
        return descs

    def transpose(b):
        for s in range(8):
            tout_s = tout.at[s]

            def tbody(k, carry, s=s):
                c0 = k * 8
                for jj in range(8):
                    c = c0 + jj
                    v0 = rows[b][s, c, pl.ds(0, 16)]
                    v1 = rows[b][s, c, pl.ds(16, 16)]
                    plsc.store_scatter(tout_s, [dbase0 + c], v0)
                    plsc.store_scatter(tout_s, [dbase1 + c], v1)
                return carry

            lax.fori_loop(0, 16, tbody, 0)

    for b in range(_NBUF):
        fire(b, b)

    def body(i, carry):
        for b in range(_NBUF):
            u = i * _NBUF + b
            wait_gathers(b)

            @pl.when(u > 0)
            def _():
                for dsc in write_descs(u - 1):
                    dsc.wait()

            transpose(b)
            for dsc in write_descs(u):
                dsc.start()

            @pl.when(i < _NGROUPS - 1)
            def _():
                fire(b, u + _NBUF)

        return carry

    lax.fori_loop(0, _NGROUPS, body, 0)
    for dsc in write_descs(_NU - 1):
        dsc.wait()


@jax.jit
def _embed(x, table):
    mesh = plsc.VectorSubcoreMesh(core_axis_name="c", subcore_axis_name="s")
    f = pl.kernel(
        _emb_body,
        out_type=jax.ShapeDtypeStruct((_L, 4, 128, 1024), jnp.float32),
        mesh=mesh,
        scratch_types=[
            pltpu.VMEM((8, 128), jnp.int32),
            pltpu.VMEM((8, 128), jnp.int32),
            pltpu.VMEM((8, 128, _D), jnp.float32),
            pltpu.VMEM((8, 128, _D), jnp.float32),
            pltpu.VMEM((8, 4096), jnp.float32),
            pltpu.SemaphoreType.DMA,
            pltpu.SemaphoreType.DMA,
            pltpu.SemaphoreType.DMA,
        ],
        compiler_params=pltpu.CompilerParams(use_tc_tiling_on_sc=False,
                                             needs_layout_passes=False),
    )
    return f(x, table)


def kernel(ys, table):
    x = jnp.transpose(
        jnp.reshape(jnp.transpose(ys.astype(jnp.int32), (1, 0)),
                    (_LT, 8, _BT, 128)),
        (0, 2, 1, 3))
    o = _embed(x, table)
    o5 = jnp.reshape(o, (_L, 4, _BT, 8, 128))
    return jnp.reshape(jnp.transpose(o5, (2, 4, 0, 1, 3)), (_B, _L, _D))


# speedup vs baseline: 1.3769x; 1.1402x over previous
import jax
import jax.numpy as jnp
from jax import lax
from jax.experimental import pallas as pl
from jax.experimental.pallas import tpu as pltpu
from jax.experimental.pallas import tpu_sc as plsc

_B, _L, _D = 16384, 200, 32
_NC, _NS = 2, 16
_NW = _NC * _NS
_LT = _L // 8
_BT = _B // 128
_NU = _LT * _BT // _NW
_NBUF = 2
_NGROUPS = _NU // _NBUF


def _emb_body(x_hbm, table_hbm, o_hbm,
              idx0, idx1, rows0, rows1, tout, gs0, gs1, ws):
    idx = (idx0, idx1)
    rows = (rows0, rows1)
    gsem = (gs0, gs1)
    wid = lax.axis_index("s") * _NC + lax.axis_index("c")
    u0 = wid * _NU
    lane = lax.iota(jnp.int32, 16)
    dbase0 = lane * 128
    dbase1 = dbase0 + 16 * 128

    def unit_lt_bt(u):
        uid = u0 + u
        return uid // _BT, uid % _BT

    def fire(b, u):
        lt, bt = unit_lt_bt(u)
        pltpu.sync_copy(x_hbm.at[lt, bt], idx[b])
        for s in range(8):
            pltpu.async_copy(table_hbm.at[idx[b].at[s]], rows[b].at[s],
                             gsem[b])

    def wait_gathers(b):
        for s in range(8):
            pltpu.make_async_copy(table_hbm.at[idx[b].at[s]], rows[b].at[s],
                                  gsem[b]).wait()

    def write_descs(u):
        lt, bt = unit_lt_bt(u)
        descs = []
        for s in range(8):
            for dt in range(4):
                descs.append(pltpu.make_async_copy(
                    tout.at[s, pl.ds(1024 * dt, 1024)],
                    o_hbm.at[8 * lt + s, dt, bt], ws))
        return descs

    def transpose(b):
        @plsc.parallel_loop(0, 128, step=1, unroll=8)
        def _(c):
            i0 = dbase0 + c
            i1 = dbase1 + c
            for s in range(8):
                v0 = rows[b][s, c, pl.ds(0, 16)]
                v1 = rows[b][s, c, pl.ds(16, 16)]
                plsc.store_scatter(tout.at[s], [i0], v0)
                plsc.store_scatter(tout.at[s], [i1], v1)

    for b in range(_NBUF):
        fire(b, b)

    def body(i, carry):
        for b in range(_NBUF):
            u = i * _NBUF + b
            wait_gathers(b)

            @pl.when(u > 0)
            def _():
                for dsc in write_descs(u - 1):
                    dsc.wait()

            transpose(b)
            for dsc in write_descs(u):
                dsc.start()

            @pl.when(i < _NGROUPS - 1)
            def _():
                fire(b, u + _NBUF)

        return carry

    lax.fori_loop(0, _NGROUPS, body, 0)
    for dsc in write_descs(_NU - 1):
        dsc.wait()


@jax.jit
def _embed(x, table):
    mesh = plsc.VectorSubcoreMesh(core_axis_name="c", subcore_axis_name="s")
    f = pl.kernel(
        _emb_body,
        out_type=jax.ShapeDtypeStruct((_L, 4, 128, 1024), jnp.float32),
        mesh=mesh,
        scratch_types=[
            pltpu.VMEM((8, 128), jnp.int32),
            pltpu.VMEM((8, 128), jnp.int32),
            pltpu.VMEM((8, 128, _D), jnp.float32),
            pltpu.VMEM((8, 128, _D), jnp.float32),
            pltpu.VMEM((8, 4096), jnp.float32),
            pltpu.SemaphoreType.DMA,
            pltpu.SemaphoreType.DMA,
            pltpu.SemaphoreType.DMA,
        ],
        compiler_params=pltpu.CompilerParams(use_tc_tiling_on_sc=False,
                                             needs_layout_passes=False),
    )
    return f(x, table)


def kernel(ys, table):
    x = jnp.transpose(
        jnp.reshape(jnp.transpose(ys.astype(jnp.int32), (1, 0)),
                    (_LT, 8, _BT, 128)),
        (0, 2, 1, 3))
    o = _embed(x, table)
    o5 = jnp.reshape(o, (_L, 4, _BT, 8, 128))
    return jnp.reshape(jnp.transpose(o5, (2, 4, 0, 1, 3)), (_B, _L, _D))


# speedup vs baseline: 3.8236x; 2.7769x over previous
import jax
import jax.numpy as jnp
from jax import lax
from jax.experimental import pallas as pl
from jax.experimental.pallas import tpu as pltpu
from jax.experimental.pallas import tpu_sc as plsc

_B, _L, _D = 16384, 200, 32
_NC, _NS = 2, 16
_NW = _NC * _NS
_LT = _L // 8
_BT = _B // 128
_NU = _LT * _BT // _NW
_NBUF = 2
_NGROUPS = _NU // _NBUF


def _emb_body(x_hbm, table_hbm, o_hbm,
              idx0, idx1, rows0, rows1, tout, gs0, gs1, ws):
    idx = (idx0, idx1)
    rows = (rows0, rows1)
    gsem = (gs0, gs1)
    wid = lax.axis_index("s") * _NC + lax.axis_index("c")
    u0 = wid * _NU
    lane = lax.iota(jnp.int32, 16)
    dvec = (lane, lane + 16)

    def unit_lt_bt(u):
        uid = u0 + u
        return uid // _BT, uid % _BT

    def fire(b, u):
        lt, bt = unit_lt_bt(u)
        pltpu.sync_copy(x_hbm.at[lt, bt], idx[b])
        for s in range(8):
            pltpu.async_copy(table_hbm.at[idx[b].at[s]], rows[b].at[s],
                             gsem[b])

    def wait_gathers(b):
        for s in range(8):
            pltpu.make_async_copy(table_hbm.at[idx[b].at[s]], rows[b].at[s],
                                  gsem[b]).wait()

    def write_descs(u):
        lt, bt = unit_lt_bt(u)
        descs = []
        for s in range(8):
            for dt in range(4):
                descs.append(pltpu.make_async_copy(
                    tout.at[s, pl.ds(8 * dt, 8)],
                    o_hbm.at[8 * lt + s, dt, bt], ws))
        return descs

    def transpose(b):
        @plsc.parallel_loop(0, 128, step=1, unroll=4)
        def _(c0):
            t = (lane + c0) & 127
            for s in range(8):
                r_s = rows[b].at[s]
                to_s = tout.at[s]
                for h in range(2):
                    v = plsc.load_gather(r_s, [t, dvec[h]])
                    plsc.store_scatter(to_s, [dvec[h], t], v)

    for b in range(_NBUF):
        fire(b, b)

    def body(i, carry):
        for b in range(_NBUF):
            u = i * _NBUF + b
            wait_gathers(b)

            @pl.when(u > 0)
            def _():
                for dsc in write_descs(u - 1):
                    dsc.wait()

            transpose(b)
            for dsc in write_descs(u):
                dsc.start()

            @pl.when(i < _NGROUPS - 1)
            def _():
                fire(b, u + _NBUF)

        return carry

    lax.fori_loop(0, _NGROUPS, body, 0)
    for dsc in write_descs(_NU - 1):
        dsc.wait()


@jax.jit
def _embed(x, table):
    mesh = plsc.VectorSubcoreMesh(core_axis_name="c", subcore_axis_name="s")
    f = pl.kernel(
        _emb_body,
        out_type=jax.ShapeDtypeStruct((_L, 4, 128, 8, 128), jnp.float32),
        mesh=mesh,
        scratch_types=[
            pltpu.VMEM((8, 128), jnp.int32),
            pltpu.VMEM((8, 128), jnp.int32),
            pltpu.VMEM((8, 128, _D), jnp.float32),
            pltpu.VMEM((8, 128, _D), jnp.float32),
            pltpu.VMEM((8, _D, 128), jnp.float32),
            pltpu.SemaphoreType.DMA,
            pltpu.SemaphoreType.DMA,
            pltpu.SemaphoreType.DMA,
        ],
        compiler_params=pltpu.CompilerParams(use_tc_tiling_on_sc=False,
                                             needs_layout_passes=False,
                                             disable_bounds_checks=True),
    )
    return f(x, table)


def kernel(ys, table):
    x = jnp.transpose(
        jnp.reshape(jnp.transpose(ys.astype(jnp.int32), (1, 0)),
                    (_LT, 8, _BT, 128)),
        (0, 2, 1, 3))
    o = _embed(x, table)
    return jnp.reshape(jnp.transpose(o, (2, 4, 0, 1, 3)), (_B, _L, _D))


# speedup vs baseline: 3.8246x; 1.0003x over previous
import jax
import jax.numpy as jnp
from jax import lax
from jax.experimental import pallas as pl
from jax.experimental.pallas import tpu as pltpu
from jax.experimental.pallas import tpu_sc as plsc

_B, _L, _D = 16384, 200, 32
_NC, _NS = 2, 16
_NW = _NC * _NS
_LT = _L // 8
_BT = _B // 128
_NU = _LT * _BT // _NW
_NBUF = 2
_NGROUPS = _NU // _NBUF


def _emb_body(x_hbm, table_hbm, o_hbm,
              idx0, idx1, rows0, rows1, tout, gs0, gs1, ws):
    idx = (idx0, idx1)
    rows = (rows0, rows1)
    gsem = (gs0, gs1)
    wid = lax.axis_index("s") * _NC + lax.axis_index("c")
    u0 = wid * _NU
    lane = lax.iota(jnp.int32, 16)
    dvec = (lane, lane + 16)

    def unit_lt_bt(u):
        uid = u0 + u
        return uid // _BT, uid % _BT

    def fire(b, u):
        lt, bt = unit_lt_bt(u)
        pltpu.sync_copy(x_hbm.at[lt, bt], idx[b])
        for s in range(8):
            pltpu.async_copy(table_hbm.at[idx[b].at[s]], rows[b].at[s],
                             gsem[b])

    def wait_gathers(b):
        for s in range(8):
            pltpu.make_async_copy(table_hbm.at[idx[b].at[s]], rows[b].at[s],
                                  gsem[b]).wait()

    def write_descs(u):
        lt, bt = unit_lt_bt(u)
        descs = []
        for s in range(8):
            for dt in range(4):
                descs.append(pltpu.make_async_copy(
                    tout.at[s, pl.ds(8 * dt, 8)],
                    o_hbm.at[8 * lt + s, dt, bt], ws))
        return descs

    def transpose(b):
        @plsc.parallel_loop(0, 128, step=1, unroll=8)
        def _(c0):
            t = (lane + c0) & 127
            for s in range(8):
                r_s = rows[b].at[s]
                to_s = tout.at[s]
                for h in range(2):
                    v = plsc.load_gather(r_s, [t, dvec[h]])
                    plsc.store_scatter(to_s, [dvec[h], t], v)

    for b in range(_NBUF):
        fire(b, b)

    def body(i, carry):
        for b in range(_NBUF):
            u = i * _NBUF + b
            wait_gathers(b)

            @pl.when(u > 0)
            def _():
                for dsc in write_descs(u - 1):
                    dsc.wait()

            transpose(b)
            for dsc in write_descs(u):
                dsc.start()

            @pl.when(i < _NGROUPS - 1)
            def _():
                fire(b, u + _NBUF)

        return carry

    lax.fori_loop(0, _NGROUPS, body, 0)
    for dsc in write_descs(_NU - 1):
        dsc.wait()


@jax.jit
def _embed(x, table):
    mesh = plsc.VectorSubcoreMesh(core_axis_name="c", subcore_axis_name="s")
    f = pl.kernel(
        _emb_body,
        out_type=jax.ShapeDtypeStruct((_L, 4, 128, 8, 128), jnp.float32),
        mesh=mesh,
        scratch_types=[
            pltpu.VMEM((8, 128), jnp.int32),
            pltpu.VMEM((8, 128), jnp.int32),
            pltpu.VMEM((8, 128, _D), jnp.float32),
            pltpu.VMEM((8, 128, _D), jnp.float32),
            pltpu.VMEM((8, _D, 128), jnp.float32),
            pltpu.SemaphoreType.DMA,
            pltpu.SemaphoreType.DMA,
            pltpu.SemaphoreType.DMA,
        ],
        compiler_params=pltpu.CompilerParams(use_tc_tiling_on_sc=False,
                                             needs_layout_passes=False,
                                             disable_bounds_checks=True),
    )
    return f(x, table)


def kernel(ys, table):
    x = jnp.transpose(
        jnp.reshape(jnp.transpose(ys.astype(jnp.int32), (1, 0)),
                    (_LT, 8, _BT, 128)),
        (0, 2, 1, 3))
    o = _embed(x, table)
    return jnp.reshape(jnp.transpose(o, (2, 4, 0, 1, 3)), (_B, _L, _D))
